# Initial kernel scaffold; baseline (speedup 1.0000x reference)
import jax
import jax.numpy as jnp
from jax.experimental import pallas as pl


def kernel(seq, qvs_idx, sum_idx, weight, bias):
    raise NotImplementedError("write your pallas kernel here")


# speedup vs baseline: 14.0418x; 14.0418x over previous
import jax
import jax.numpy as jnp
from jax import lax
from jax.experimental import pallas as pl

N = 1024
D = 128


def _cov_kernel(seq_ref, q_ref, u_ref, w_ref, b_ref, out_ref):
    s = seq_ref[:]
    q = q_ref[:]
    u = u_ref[:]

    a = s * q
    b = s * u

    dn = (((1,), (1,)), ((), ()))
    g = lax.dot_general(a, b, dn, preferred_element_type=jnp.float32)

    ra = jnp.sum(a * a, axis=1, keepdims=True)
    rs = jnp.sum(s * s, axis=1, keepdims=True)

    ones_row = jnp.ones((1, D), dtype=jnp.float32)
    bb = b * b
    rb_t = lax.dot_general(ones_row, bb, dn, preferred_element_type=jnp.float32)
    ones_1 = jnp.ones((1, 1), dtype=jnp.float32)
    uu_t = lax.dot_general(ones_1, u * u, dn, preferred_element_type=jnp.float32)

    d2 = jnp.maximum(ra + rb_t - 2.0 * g, 0.0)
    d = jnp.sqrt(d2)

    diag = jnp.abs(q - u) * jnp.sqrt(rs)
    row_i = lax.broadcasted_iota(jnp.int32, (N, N), 0)
    col_i = lax.broadcasted_iota(jnp.int32, (N, N), 1)
    d = jnp.where(row_i == col_i, diag, d)

    norm = jnp.mean(d)

    masked = jnp.where(uu_t > 0.0, d, jnp.inf)
    dmin = jnp.min(masked, axis=1, keepdims=True)
    dmin = jnp.where(dmin > norm, norm, dmin)
    simcov = 1.0 - dmin / norm

    out_ref[:] = simcov * w_ref[0, 0] + b_ref[0, 0]


def kernel(seq, qvs_idx, sum_idx, weight, bias):
    out = pl.pallas_call(
        _cov_kernel,
        out_shape=jax.ShapeDtypeStruct((N, 1), jnp.float32),
    )(seq, qvs_idx, sum_idx, weight, bias.reshape(1, 1))
    return out
